# Initial kernel scaffold; baseline (speedup 1.0000x reference)
import functools

import numpy as np
import jax
import jax.numpy as jnp
from jax import lax
from jax.experimental import pallas as pl
from jax.experimental.pallas import tpu as pltpu
from jax.experimental.pallas import tpu_sc as plsc

_MAX_SEQ = 200
_D = 64
_BATCH = 4096
_SEQ = 200

_info = plsc.get_sparse_core_info()
_NC = _info.num_cores
_NS = _info.num_subcores
_NW = _NC * _NS

_N_ROWS = _BATCH * _SEQ
_ROWS_PER_W = _N_ROWS // _NW
_CHUNK = _SEQ
_N_CHUNKS = _ROWS_PER_W // _CHUNK
_GSUB = 40
_NSUB = _CHUNK // _GSUB
_LANES = 16


def _pe_table() -> np.ndarray:
    row_vec = np.zeros(_D, dtype=np.float64)
    row_vec[::2] = np.arange(0, _D, 2) / _D
    row_vec[1::2] = np.arange(0, _D, 2) / _D
    row_vec = 10000.0 ** row_vec
    col_vec = np.arange(0, _MAX_SEQ, 1, dtype=np.float64).reshape(-1, 1)
    pe = col_vec / row_vec
    pe[:, ::2] = np.sin(pe[:, ::2])
    pe[:, 1::2] = np.cos(pe[:, 1::2])
    return pe.astype(np.float32)


@functools.partial(
    pl.kernel,
    mesh=plsc.VectorSubcoreMesh(core_axis_name="c", subcore_axis_name="s"),
    out_type=jax.ShapeDtypeStruct((_N_ROWS, _D), jnp.float32),
    scratch_types=[
        pltpu.VMEM((_ROWS_PER_W,), jnp.int32),
        pltpu.VMEM((_SEQ, _D), jnp.float32),
        pltpu.VMEM((_CHUNK, _D), jnp.float32),
        pltpu.SemaphoreType.DMA,
    ],
)
def _sc_embed(x_hbm, table_hbm, pe_hbm, out_hbm, idx_v, pe_v, rows_v, sem):
    wid = lax.axis_index("s") * _NC + lax.axis_index("c")
    base = wid * _ROWS_PER_W
    pltpu.sync_copy(x_hbm.at[pl.ds(base, _ROWS_PER_W)], idx_v)
    pltpu.sync_copy(pe_hbm, pe_v)

    def chunk_body(g, carry):
        r0 = g * _CHUNK
        copies = []
        for j in range(_NSUB):
            copies.append(
                pltpu.async_copy(
                    table_hbm.at[idx_v.at[pl.ds(r0 + j * _GSUB, _GSUB)]],
                    rows_v.at[pl.ds(j * _GSUB, _GSUB)],
                    sem,
                )
            )
        for c in copies:
            c.wait()

        def add_body(r, carry2):
            for c in range(_D // _LANES):
                col = pl.ds(c * _LANES, _LANES)
                rows_v[r, col] = rows_v[r, col] + pe_v[r, col]
            return carry2

        lax.fori_loop(0, _CHUNK, add_body, 0)
        pltpu.sync_copy(rows_v, out_hbm.at[pl.ds(base + r0, _CHUNK)])
        return carry

    lax.fori_loop(0, _N_CHUNKS, chunk_body, 0)


def kernel(x, table):
    pe = jnp.asarray(_pe_table())
    out = _sc_embed(x.reshape(-1), table, pe)
    return out.reshape(_BATCH, _SEQ, _D)


# speedup vs baseline: 3.3036x; 3.3036x over previous
import functools

import numpy as np
import jax
import jax.numpy as jnp
from jax import lax
from jax.experimental import pallas as pl
from jax.experimental.pallas import tpu as pltpu
from jax.experimental.pallas import tpu_sc as plsc

_MAX_SEQ = 200
_D = 64
_BATCH = 4096
_SEQ = 200

_info = plsc.get_sparse_core_info()
_NC = _info.num_cores
_NS = _info.num_subcores
_NW = _NC * _NS

_N_ROWS = _BATCH * _SEQ
_ROWS_PER_W = _N_ROWS // _NW
_CHUNK = _SEQ
_N_CHUNKS = _ROWS_PER_W // _CHUNK
_GSUB = 40
_NSUB = _CHUNK // _GSUB
_LANES = 16


def _pe_table() -> np.ndarray:
    row_vec = np.zeros(_D, dtype=np.float64)
    row_vec[::2] = np.arange(0, _D, 2) / _D
    row_vec[1::2] = np.arange(0, _D, 2) / _D
    row_vec = 10000.0 ** row_vec
    col_vec = np.arange(0, _MAX_SEQ, 1, dtype=np.float64).reshape(-1, 1)
    pe = col_vec / row_vec
    pe[:, ::2] = np.sin(pe[:, ::2])
    pe[:, 1::2] = np.cos(pe[:, 1::2])
    return pe.astype(np.float32)


@functools.partial(
    pl.kernel,
    mesh=plsc.VectorSubcoreMesh(core_axis_name="c", subcore_axis_name="s"),
    out_type=jax.ShapeDtypeStruct((_N_ROWS, _D), jnp.float32),
    compiler_params=pltpu.CompilerParams(use_tc_tiling_on_sc=False),
    scratch_types=[
        pltpu.VMEM((_ROWS_PER_W,), jnp.int32),
        pltpu.VMEM((_SEQ, _D), jnp.float32),
        pltpu.VMEM((_CHUNK, _D), jnp.float32),
        pltpu.SemaphoreType.DMA,
    ],
)
def _sc_embed(x_hbm, table_hbm, pe_hbm, out_hbm, idx_v, pe_v, rows_v, sem):
    wid = lax.axis_index("s") * _NC + lax.axis_index("c")
    base = wid * _ROWS_PER_W
    pltpu.sync_copy(x_hbm.at[pl.ds(base, _ROWS_PER_W)], idx_v)
    pltpu.sync_copy(pe_hbm, pe_v)

    def chunk_body(g, carry):
        r0 = g * _CHUNK
        copies = []
        for j in range(_NSUB):
            copies.append(
                pltpu.async_copy(
                    table_hbm.at[idx_v.at[pl.ds(r0 + j * _GSUB, _GSUB)]],
                    rows_v.at[pl.ds(j * _GSUB, _GSUB)],
                    sem,
                )
            )
        for c in copies:
            c.wait()

        def add_body(r, carry2):
            for c in range(_D // _LANES):
                col = pl.ds(c * _LANES, _LANES)
                rows_v[r, col] = rows_v[r, col] + pe_v[r, col]
            return carry2

        lax.fori_loop(0, _CHUNK, add_body, 0)
        pltpu.sync_copy(rows_v, out_hbm.at[pl.ds(base + r0, _CHUNK)])
        return carry

    lax.fori_loop(0, _N_CHUNKS, chunk_body, 0)


def kernel(x, table):
    pe = jnp.asarray(_pe_table())
    out = _sc_embed(x.reshape(-1), table, pe)
    return out.reshape(_BATCH, _SEQ, _D)


# speedup vs baseline: 4.2297x; 1.2803x over previous
import functools

import numpy as np
import jax
import jax.numpy as jnp
from jax import lax
from jax.experimental import pallas as pl
from jax.experimental.pallas import tpu as pltpu
from jax.experimental.pallas import tpu_sc as plsc

_MAX_SEQ = 200
_D = 64
_BATCH = 4096
_SEQ = 200

_info = plsc.get_sparse_core_info()
_NC = _info.num_cores
_NS = _info.num_subcores
_NW = _NC * _NS

_N_ROWS = _BATCH * _SEQ
_ROWS_PER_W = _N_ROWS // _NW
_CHUNK = 2 * _SEQ
_N_CHUNKS = _ROWS_PER_W // _CHUNK
_GSUB = 80
_NSUB = _CHUNK // _GSUB
_LANES = 16
_NBUF = 3


def _pe_table() -> np.ndarray:
    row_vec = np.zeros(_D, dtype=np.float64)
    row_vec[::2] = np.arange(0, _D, 2) / _D
    row_vec[1::2] = np.arange(0, _D, 2) / _D
    row_vec = 10000.0 ** row_vec
    col_vec = np.arange(0, _MAX_SEQ, 1, dtype=np.float64).reshape(-1, 1)
    pe = col_vec / row_vec
    pe[:, ::2] = np.sin(pe[:, ::2])
    pe[:, 1::2] = np.cos(pe[:, 1::2])
    return np.tile(pe.astype(np.float32), (_CHUNK // _MAX_SEQ, 1))


@functools.partial(
    pl.kernel,
    mesh=plsc.VectorSubcoreMesh(core_axis_name="c", subcore_axis_name="s"),
    out_type=jax.ShapeDtypeStruct((_N_ROWS, _D), jnp.float32),
    compiler_params=pltpu.CompilerParams(use_tc_tiling_on_sc=False),
    scratch_types=[
        pltpu.VMEM((_ROWS_PER_W,), jnp.int32),
        pltpu.VMEM((_CHUNK, _D), jnp.float32),
        pltpu.VMEM((_CHUNK, _D), jnp.float32),
        pltpu.VMEM((_CHUNK, _D), jnp.float32),
        pltpu.VMEM((_CHUNK, _D), jnp.float32),
        pltpu.SemaphoreType.DMA,
        pltpu.SemaphoreType.DMA,
        pltpu.SemaphoreType.DMA,
        pltpu.SemaphoreType.DMA,
        pltpu.SemaphoreType.DMA,
        pltpu.SemaphoreType.DMA,
    ],
)
def _sc_embed(x_hbm, table_hbm, pe_hbm, out_hbm, idx_v, pe_v,
              b0, b1, b2, g0, g1, g2, s0, s1, s2):
    bufs = (b0, b1, b2)
    gsems = (g0, g1, g2)
    ssems = (s0, s1, s2)
    wid = lax.axis_index("s") * _NC + lax.axis_index("c")
    base = wid * _ROWS_PER_W
    pltpu.sync_copy(x_hbm.at[pl.ds(base, _ROWS_PER_W)], idx_v)
    pltpu.sync_copy(pe_hbm, pe_v)

    def issue_gather(g, k):
        r0 = g * _CHUNK
        for j in range(_NSUB):
            pltpu.async_copy(
                table_hbm.at[idx_v.at[pl.ds(r0 + j * _GSUB, _GSUB)]],
                bufs[k].at[pl.ds(j * _GSUB, _GSUB)],
                gsems[k],
            )

    def drain_gather(g, k):
        r0 = g * _CHUNK
        for j in range(_NSUB):
            pltpu.make_async_copy(
                table_hbm.at[idx_v.at[pl.ds(r0 + j * _GSUB, _GSUB)]],
                bufs[k].at[pl.ds(j * _GSUB, _GSUB)],
                gsems[k],
            ).wait()

    def add_pe(k):
        buf = bufs[k]

        def body(r, carry):
            for c in range(_D // _LANES):
                col = pl.ds(c * _LANES, _LANES)
                buf[r, col] = buf[r, col] + pe_v[r, col]
            return carry

        lax.fori_loop(0, _CHUNK, body, 0)

    def issue_store(g, k):
        pltpu.async_copy(
            bufs[k], out_hbm.at[pl.ds(base + g * _CHUNK, _CHUNK)], ssems[k])

    def wait_store(g_prev, k):
        pltpu.make_async_copy(
            bufs[k], out_hbm.at[pl.ds(base + g_prev * _CHUNK, _CHUNK)],
            ssems[k],
        ).wait()

    def process(g, k, store_wait, issue_next):
        drain_gather(g, k)
        add_pe(k)
        if store_wait:
            wait_store(g - 1, (k + 2) % _NBUF)
        if issue_next:
            issue_gather(g + 2, (k + 2) % _NBUF)
        issue_store(g, k)

    issue_gather(0, 0)
    issue_gather(1, 1)
    process(0, 0, store_wait=False, issue_next=True)
    process(1, 1, store_wait=True, issue_next=True)

    n_trips = (_N_CHUNKS - 4) // _NBUF

    def trip(i, carry):
        gbase = 2 + i * _NBUF
        for k0 in range(_NBUF):
            process(gbase + k0, (2 + k0) % _NBUF, store_wait=True,
                    issue_next=True)
        return carry

    lax.fori_loop(0, n_trips, trip, 0)
    process(_N_CHUNKS - 2, (_N_CHUNKS - 2) % _NBUF, store_wait=True,
            issue_next=False)
    process(_N_CHUNKS - 1, (_N_CHUNKS - 1) % _NBUF, store_wait=True,
            issue_next=False)
    wait_store(_N_CHUNKS - 1, (_N_CHUNKS - 1) % _NBUF)


def kernel(x, table):
    pe = jnp.asarray(_pe_table())
    out = _sc_embed(x.reshape(-1), table, pe)
    return out.reshape(_BATCH, _SEQ, _D)
